# Initial kernel scaffold; baseline (speedup 1.0000x reference)
import jax
import jax.numpy as jnp
from jax.experimental import pallas as pl


def kernel(x, edge_index, batch, W1, b1, Wg1, bg1, Wg2, bg2, Wg3, bg3, W2, b2):
    raise NotImplementedError("write your pallas kernel here")


# speedup vs baseline: 17.8301x; 17.8301x over previous
import functools

import jax
import jax.numpy as jnp
from jax import lax
from jax.experimental import pallas as pl
from jax.experimental.pallas import tpu as pltpu
from jax.experimental.pallas import tpu_sc as plsc

_N = 10000
_E = 320000
_D = 128
_G = 64
_NC = 2
_NS = 16
_NW = _NC * _NS
_EPW = _E // _NW
_K = 125
_NCH = _EPW // _K
_NP = 10112
_RPT = _NP // _NS
_BLK = 1000
_NBLK = _N // _BLK


def _sc_mesh():
    return plsc.VectorSubcoreMesh(
        core_axis_name="c", subcore_axis_name="s",
        num_cores=_NC, num_subcores=_NS)


def _sc_degree(dst_r, ones_rows, zeros128):
    @functools.partial(
        pl.kernel,
        out_type=jax.ShapeDtypeStruct((_NC, _NP, _D), jnp.float32),
        mesh=_sc_mesh(),
        scratch_types=[
            pltpu.VMEM((_NCH, _K), jnp.int32),
            pltpu.VMEM((_K, _D), jnp.float32),
            pltpu.VMEM_SHARED((_NP, _D), jnp.float32),
        ],
    )
    def deg_kernel(dst_hbm, ones_hbm, zeros_hbm, out_hbm, dst_v, ones_v, deg_sh):
        c = lax.axis_index("c")
        s = lax.axis_index("s")
        w = s * _NC + c
        pltpu.sync_copy(dst_hbm.at[w], dst_v)
        pltpu.sync_copy(ones_hbm, ones_v)
        pltpu.sync_copy(zeros_hbm, deg_sh.at[pl.ds(s * _RPT, _RPT)])
        plsc.subcore_barrier()

        def body(j, carry):
            pltpu.sync_copy(ones_v, deg_sh.at[dst_v.at[j]], add=True)
            return carry

        lax.fori_loop(0, _NCH, body, 0)
        plsc.subcore_barrier()
        pltpu.sync_copy(deg_sh.at[pl.ds(s * _RPT, _RPT)],
                        out_hbm.at[c, pl.ds(s * _RPT, _RPT)])

    return deg_kernel(dst_r, ones_rows, zeros128)


def _sc_scatter(y, src_r, dst_r, zeros128):
    @functools.partial(
        pl.kernel,
        out_type=jax.ShapeDtypeStruct((_NC, _NP, _D), jnp.float32),
        mesh=_sc_mesh(),
        scratch_types=[
            pltpu.VMEM((_NCH, _K), jnp.int32),
            pltpu.VMEM((_NCH, _K), jnp.int32),
            pltpu.VMEM((_K, _D), jnp.float32),
            pltpu.SemaphoreType.DMA,
            pltpu.VMEM_SHARED((_NP, _D), jnp.float32),
        ],
    )
    def scat_kernel(y_hbm, src_hbm, dst_hbm, zeros_hbm, out_hbm,
                    src_v, dst_v, rows_v, sem, z_sh):
        c = lax.axis_index("c")
        s = lax.axis_index("s")
        w = s * _NC + c
        pltpu.sync_copy(src_hbm.at[w], src_v)
        pltpu.sync_copy(dst_hbm.at[w], dst_v)
        pltpu.sync_copy(zeros_hbm, z_sh.at[pl.ds(s * _RPT, _RPT)])
        plsc.subcore_barrier()

        def body(j, carry):
            pltpu.async_copy(y_hbm.at[src_v.at[j]], rows_v, sem).wait()
            pltpu.sync_copy(rows_v, z_sh.at[dst_v.at[j]], add=True)
            return carry

        lax.fori_loop(0, _NCH, body, 0)
        plsc.subcore_barrier()
        pltpu.sync_copy(z_sh.at[pl.ds(s * _RPT, _RPT)],
                        out_hbm.at[c, pl.ds(s * _RPT, _RPT)])

    return scat_kernel(y, src_r, dst_r, zeros128)


def _dinv_block(deg_ref):
    return lax.rsqrt(deg_ref[0, :, 0:1] + deg_ref[1, :, 0:1] + 1.0)


def _tc_dense1(x, deg, W1, b1, Wg1):
    def body(x_ref, deg_ref, W1_ref, b1_ref, Wg1_ref, y_ref):
        dinv = _dinv_block(deg_ref)
        h = jnp.maximum(
            jnp.dot(x_ref[...], W1_ref[...],
                    preferred_element_type=jnp.float32) + b1_ref[...], 0.0)
        y_ref[...] = dinv * jnp.dot(h, Wg1_ref[...],
                                    preferred_element_type=jnp.float32)

    return pl.pallas_call(
        body,
        grid=(_NBLK,),
        in_specs=[
            pl.BlockSpec((_BLK, _D), lambda i: (i, 0)),
            pl.BlockSpec((_NC, _BLK, _D), lambda i: (0, i, 0)),
            pl.BlockSpec((_D, _D), lambda i: (0, 0)),
            pl.BlockSpec((1, _D), lambda i: (0, 0)),
            pl.BlockSpec((_D, _D), lambda i: (0, 0)),
        ],
        out_specs=pl.BlockSpec((_BLK, _D), lambda i: (i, 0)),
        out_shape=jax.ShapeDtypeStruct((_N, _D), jnp.float32),
    )(x, deg, W1, b1, Wg1)


def _tc_mid(p, yprev, deg, bg, Wgn):
    def body(p_ref, y_ref, deg_ref, bg_ref, Wg_ref, o_ref):
        dinv = _dinv_block(deg_ref)
        z = p_ref[0] + p_ref[1] + y_ref[...]
        h = jnp.maximum(dinv * z + bg_ref[...], 0.0)
        o_ref[...] = dinv * jnp.dot(h, Wg_ref[...],
                                    preferred_element_type=jnp.float32)

    return pl.pallas_call(
        body,
        grid=(_NBLK,),
        in_specs=[
            pl.BlockSpec((_NC, _BLK, _D), lambda i: (0, i, 0)),
            pl.BlockSpec((_BLK, _D), lambda i: (i, 0)),
            pl.BlockSpec((_NC, _BLK, _D), lambda i: (0, i, 0)),
            pl.BlockSpec((1, _D), lambda i: (0, 0)),
            pl.BlockSpec((_D, _D), lambda i: (0, 0)),
        ],
        out_specs=pl.BlockSpec((_BLK, _D), lambda i: (i, 0)),
        out_shape=jax.ShapeDtypeStruct((_N, _D), jnp.float32),
    )(p, yprev, deg, bg, Wgn)


def _tc_pool(p, y3, deg, bg3, batch_r, W2, b2):
    def body(p_ref, y_ref, deg_ref, bg_ref, b_ref, W2_ref, b2_ref,
             o_ref, sums, cnts):
        i = pl.program_id(0)

        @pl.when(i == 0)
        def _():
            sums[...] = jnp.zeros_like(sums)
            cnts[...] = jnp.zeros_like(cnts)

        dinv = _dinv_block(deg_ref)
        h = jnp.maximum(dinv * (p_ref[0] + p_ref[1] + y_ref[...])
                        + bg_ref[...], 0.0)
        gids = lax.broadcasted_iota(jnp.int32, (_G, _BLK), 0)
        onehot = (gids == b_ref[0]).astype(jnp.float32)
        sums[...] += jnp.dot(onehot, h, preferred_element_type=jnp.float32)
        cnts[...] += jnp.broadcast_to(
            jnp.sum(onehot, axis=1, keepdims=True), (_G, _D))

        @pl.when(i == _NBLK - 1)
        def _():
            pooled = sums[...] / jnp.maximum(cnts[...], 1.0)
            o_ref[...] = jnp.dot(pooled, W2_ref[...],
                                 preferred_element_type=jnp.float32) + b2_ref[...]

    return pl.pallas_call(
        body,
        grid=(_NBLK,),
        in_specs=[
            pl.BlockSpec((_NC, _BLK, _D), lambda i: (0, i, 0)),
            pl.BlockSpec((_BLK, _D), lambda i: (i, 0)),
            pl.BlockSpec((_NC, _BLK, _D), lambda i: (0, i, 0)),
            pl.BlockSpec((1, _D), lambda i: (0, 0)),
            pl.BlockSpec((1, 1, _BLK), lambda i: (i, 0, 0)),
            pl.BlockSpec((_D, _D), lambda i: (0, 0)),
            pl.BlockSpec((1, _D), lambda i: (0, 0)),
        ],
        out_specs=pl.BlockSpec((_G, _D), lambda i: (0, 0)),
        out_shape=jax.ShapeDtypeStruct((_G, _D), jnp.float32),
        scratch_shapes=[
            pltpu.VMEM((_G, _D), jnp.float32),
            pltpu.VMEM((_G, _D), jnp.float32),
        ],
    )(p, y3, deg, bg3, batch_r, W2, b2)


def kernel(x, edge_index, batch, W1, b1, Wg1, bg1, Wg2, bg2, Wg3, bg3, W2, b2):
    src = edge_index[0].reshape(_NW, _NCH, _K)
    dst = edge_index[1].reshape(_NW, _NCH, _K)
    zeros128 = jnp.zeros((_RPT, _D), jnp.float32)
    ones128 = jnp.ones((_K, _D), jnp.float32)
    batch_r = batch.reshape(_NBLK, 1, _BLK)
    b1r = b1.reshape(1, _D)
    bg1r = bg1.reshape(1, _D)
    bg2r = bg2.reshape(1, _D)
    bg3r = bg3.reshape(1, _D)
    b2r = b2.reshape(1, _D)

    deg = _sc_degree(dst, ones128, zeros128)
    y1 = _tc_dense1(x, deg, W1, b1r, Wg1)
    p1 = _sc_scatter(y1, src, dst, zeros128)
    y2 = _tc_mid(p1, y1, deg, bg1r, Wg2)
    p2 = _sc_scatter(y2, src, dst, zeros128)
    y3 = _tc_mid(p2, y2, deg, bg2r, Wg3)
    p3 = _sc_scatter(y3, src, dst, zeros128)
    return _tc_pool(p3, y3, deg, bg3r, batch_r, W2, b2r)
